# baseline (speedup 1.0000x reference)
import functools

import jax
import jax.numpy as jnp
from jax import lax
from jax.experimental import pallas as pl
from jax.experimental.pallas import tpu as pltpu
from jax.experimental.pallas import tpu_sc as plsc

N_NODES = 10000
D = 128
N_EDGES = 320000
NC = 2
NS = 16
NW = NC * NS
L = 16
CH = 128
NBUF = 2
NCHUNK = 80
EPT = NCHUNK * CH
E_PAD = NW * EPT
P = 10112
RPT = P // NS


NIB = 2 * NBUF


def _sc_scatter(x, e3, zeros2, zeros1):
    mesh = plsc.VectorSubcoreMesh(core_axis_name="c", subcore_axis_name="s")

    @functools.partial(
        pl.kernel,
        mesh=mesh,
        out_type=[
            jax.ShapeDtypeStruct((NC, P, D), jnp.float32),
            jax.ShapeDtypeStruct((NW, P), jnp.float32),
        ],
        scratch_types=[
            pltpu.VMEM_SHARED((P, D), jnp.float32),
            pltpu.VMEM((P,), jnp.float32),
        ] + [pltpu.VMEM((2, CH), jnp.int32)] * NIB
          + [pltpu.VMEM((CH, D), jnp.float32)] * NBUF
          + [pltpu.SemaphoreType.DMA] * (NIB + 2 * NBUF),
        compiler_params=pltpu.CompilerParams(needs_layout_passes=False),
    )
    def k(x_hbm, e_hbm, z2_hbm, z1_hbm, psum_hbm, degs_hbm,
          acc, degb, *bufs):
        idxbs = bufs[:NIB]
        rowbs = bufs[NIB:NIB + NBUF]
        isems = bufs[NIB + NBUF:2 * NIB + NBUF]
        gsems = bufs[2 * NIB + NBUF:2 * NIB + 2 * NBUF]
        ssems = bufs[2 * NIB + 2 * NBUF:]
        c = lax.axis_index("c")
        s = lax.axis_index("s")
        wid = s * NC + c
        for i in range(NIB):
            pltpu.async_copy(e_hbm.at[wid, i], idxbs[i], isems[i])
        for b in range(NBUF):
            pltpu.make_async_copy(e_hbm.at[wid, b], idxbs[b], isems[b]).wait()
            pltpu.async_copy(x_hbm.at[idxbs[b].at[0]], rowbs[b], gsems[b])
        pltpu.sync_copy(z2_hbm.at[pl.ds(s * RPT, RPT)],
                        acc.at[pl.ds(s * RPT, RPT)])
        pltpu.sync_copy(z1_hbm, degb)
        plsc.subcore_barrier()

        ones = jnp.full((L,), 1.0, jnp.float32)

        def block(go, carry):
            for i in range(NIB):
                g = go * NIB + i
                b = i % NBUF
                pltpu.make_async_copy(
                    x_hbm.at[pl.ds(0, CH)], rowbs[b], gsems[b]).wait()
                scat = pltpu.async_copy(
                    rowbs[b], acc.at[idxbs[i].at[1]], ssems[b], add=True)
                for j in range(CH // L):
                    idx = idxbs[i][1, pl.ds(j * L, L)]
                    plsc.addupdate_scatter(degb, [idx], ones)
                scat.wait()
                i2 = (i + NBUF) % NIB
                pltpu.make_async_copy(
                    e_hbm.at[wid, 0], idxbs[i2], isems[i2]).wait()
                pltpu.async_copy(x_hbm.at[idxbs[i2].at[0]], rowbs[b], gsems[b])
                nxt = lax.rem(g + NIB, NCHUNK)
                pltpu.async_copy(e_hbm.at[wid, nxt], idxbs[i], isems[i])
            return carry

        lax.fori_loop(0, NCHUNK // NIB, block, 0)
        for b in range(NBUF):
            pltpu.make_async_copy(
                x_hbm.at[pl.ds(0, CH)], rowbs[b], gsems[b]).wait()
        for i in range(NBUF, NIB):
            pltpu.make_async_copy(
                e_hbm.at[wid, 0], idxbs[i], isems[i]).wait()
        plsc.subcore_barrier()
        pltpu.sync_copy(acc.at[pl.ds(s * RPT, RPT)],
                        psum_hbm.at[c, pl.ds(s * RPT, RPT)])
        pltpu.sync_copy(degb, degs_hbm.at[wid])

    return k(x, e3, zeros2, zeros1)


BR = 128


def _combine(psum, degs):
    def body(p_ref, d_ref, o_ref):
        p = p_ref[...]
        d = jnp.sum(d_ref[...], axis=0)
        o_ref[...] = (p[0] + p[1]) / jnp.maximum(d, 1.0)[:, None]

    return pl.pallas_call(
        body,
        grid=(P // BR,),
        in_specs=[
            pl.BlockSpec((NC, BR, D), lambda i: (0, i, 0)),
            pl.BlockSpec((NW, BR), lambda i: (0, i)),
        ],
        out_specs=pl.BlockSpec((BR, D), lambda i: (i, 0)),
        out_shape=jax.ShapeDtypeStruct((P, D), jnp.float32),
    )(psum, degs)


def kernel(x, edge_index):
    ei = edge_index.astype(jnp.int32)
    pad = E_PAD - N_EDGES
    src = jnp.pad(ei[0], (0, pad)).reshape(NW, NCHUNK, 1, CH)
    dst = jnp.pad(ei[1], (0, pad), constant_values=N_NODES).reshape(NW, NCHUNK, 1, CH)
    e3 = jnp.concatenate([src, dst], axis=2)
    zeros2 = jnp.zeros((P, D), jnp.float32)
    zeros1 = jnp.zeros((P,), jnp.float32)
    psum, degs = _sc_scatter(x, e3, zeros2, zeros1)
    return _combine(psum, degs)[:N_NODES]
